# baseline (speedup 1.0000x reference)
import dataclasses
import functools

import jax
import jax.numpy as jnp
from jax.experimental import pallas as pl
from jax.experimental.pallas import tpu as pltpu
from jax.experimental.pallas import tpu_sc as plsc

HIDDEN = 1024
FFN = 2048
NUM_EXPERTS = 16
TOP_K = 2
CHUNK = 1024
N_CHUNKS = FFN // CHUNK

SC_LANES = 16
K_CHUNKS = HIDDEN // SC_LANES
TOKENS_PER_SUBCORE = 4


def _sc_logits_body(x_vmem, gw_vmem, out_vmem):
    @pl.loop(0, TOKENS_PER_SUBCORE)
    def _(r):
        row = jnp.zeros((SC_LANES,), jnp.float32)
        lane = jax.lax.iota(jnp.int32, SC_LANES)
        for e in range(NUM_EXPERTS):
            acc = jnp.zeros((SC_LANES,), jnp.float32)
            for k in range(K_CHUNKS):
                xv = x_vmem[pl.ds(r * HIDDEN + k * SC_LANES, SC_LANES)]
                gv = gw_vmem[pl.ds(e * HIDDEN + k * SC_LANES, SC_LANES)]
                acc = acc + xv * gv
            s = jnp.sum(acc)
            row = jnp.where(lane == e, s, row)
        out_vmem[pl.ds(r * NUM_EXPERTS, NUM_EXPERTS)] = row


def _sc_logits(x, gate_w):
    t = x.shape[0]
    xr = x.reshape(t * HIDDEN)
    gwr = gate_w.reshape(NUM_EXPERTS * HIDDEN)
    n_units = t // TOKENS_PER_SUBCORE

    sc_params = pltpu.CompilerParams()
    if "needs_layout_passes" in pltpu.CompilerParams.__dataclass_fields__:
        sc_params = dataclasses.replace(sc_params, needs_layout_passes=False)

    @functools.partial(
        pl.kernel,
        out_type=jax.ShapeDtypeStruct((t * NUM_EXPERTS,), jnp.float32),
        mesh=plsc.VectorSubcoreMesh(core_axis_name="c", subcore_axis_name="s"),
        compiler_params=sc_params,
    )
    def run(x_hbm, gw_hbm, out_hbm):
        pltpu.emit_pipeline(
            _sc_logits_body,
            grid=(n_units,),
            in_specs=[
                pl.BlockSpec((TOKENS_PER_SUBCORE * HIDDEN,), lambda i: (i,)),
                pl.BlockSpec((NUM_EXPERTS * HIDDEN,), lambda i: (0,)),
            ],
            out_specs=[
                pl.BlockSpec((TOKENS_PER_SUBCORE * NUM_EXPERTS,),
                             lambda i: (i,)),
            ],
            core_axis_name=("c", "s"),
            dimension_semantics=(pltpu.PARALLEL,),
        )(x_hbm, gw_hbm, out_hbm)

    return run(xr, gwr).reshape(t, NUM_EXPERTS)


def _moe_kernel(x_ref, gw_ref, wg_ref, wu_ref, wd_ref, out_ref, combine_ref):
    e = pl.program_id(0)
    c = pl.program_id(1)
    first = jnp.logical_and(e == 0, c == 0)

    @pl.when(first)
    def _router():
        x = x_ref[...]
        logits = jax.lax.dot_general(
            x, gw_ref[...], dimension_numbers=(((1,), (1,)), ((), ())),
            preferred_element_type=jnp.float32)
        probs = jax.nn.softmax(logits, axis=-1)
        eidx = jax.lax.broadcasted_iota(jnp.int32, probs.shape, 1)
        p1 = jnp.max(probs, axis=-1, keepdims=True)
        i1 = jnp.min(jnp.where(probs >= p1, eidx, NUM_EXPERTS), axis=-1,
                     keepdims=True)
        sel1 = eidx == i1
        probs2 = jnp.where(sel1, -jnp.inf, probs)
        p2 = jnp.max(probs2, axis=-1, keepdims=True)
        i2 = jnp.min(jnp.where(probs2 >= p2, eidx, NUM_EXPERTS), axis=-1,
                     keepdims=True)
        sel2 = eidx == i2
        denom = p1 + p2
        combine_ref[...] = (jnp.where(sel1, p1, 0.0)
                            + jnp.where(sel2, p2, 0.0)) / denom

    x = x_ref[...].astype(jnp.bfloat16)
    gate = jnp.dot(x, wg_ref[0].astype(jnp.bfloat16),
                   preferred_element_type=jnp.float32)
    up = jnp.dot(x, wu_ref[0].astype(jnp.bfloat16),
                 preferred_element_type=jnp.float32)
    hidden = gate * jax.nn.sigmoid(gate) * up
    down = jnp.dot(hidden.astype(jnp.bfloat16), wd_ref[0].astype(jnp.bfloat16),
                   preferred_element_type=jnp.float32)
    combine = combine_ref[...]
    lane = jax.lax.broadcasted_iota(jnp.int32, combine.shape, 1)
    col = jnp.sum(jnp.where(lane == e, combine, 0.0), axis=-1, keepdims=True)
    contrib = col * down

    @pl.when(first)
    def _init():
        out_ref[...] = contrib

    @pl.when(jnp.logical_not(first))
    def _acc():
        out_ref[...] = out_ref[...] + contrib


@functools.partial(jax.jit, static_argnames=())
def kernel(hidden_states, gate_w, w_gate_up, w_down):
    b, s, d = hidden_states.shape
    t = b * s
    x = hidden_states.reshape(t, d)

    out = pl.pallas_call(
        _moe_kernel,
        grid=(NUM_EXPERTS, N_CHUNKS),
        in_specs=[
            pl.BlockSpec((t, d), lambda e, c: (0, 0)),
            pl.BlockSpec((NUM_EXPERTS, d), lambda e, c: (0, 0)),
            pl.BlockSpec((1, d, CHUNK), lambda e, c: (e, 0, c)),
            pl.BlockSpec((1, d, CHUNK), lambda e, c: (e, 0, N_CHUNKS + c)),
            pl.BlockSpec((1, CHUNK, d), lambda e, c: (e, c, 0)),
        ],
        out_specs=pl.BlockSpec((t, d), lambda e, c: (0, 0)),
        out_shape=jax.ShapeDtypeStruct((t, d), jnp.float32),
        scratch_shapes=[pltpu.VMEM((t, NUM_EXPERTS), jnp.float32)],
        compiler_params=pltpu.CompilerParams(
            dimension_semantics=("arbitrary", "arbitrary"),
        ),
    )(x, gate_w, w_gate_up, w_gate_up, w_down)

    logits = _sc_logits(x, gate_w)

    return out.reshape(b, s, d), logits


# speedup vs baseline: 1.1560x; 1.1560x over previous
import functools

import jax
import jax.numpy as jnp
from jax.experimental import pallas as pl
from jax.experimental.pallas import tpu as pltpu

HIDDEN = 1024
FFN = 2048
NUM_EXPERTS = 16
TOP_K = 2
CHUNK = 1024
N_CHUNKS = FFN // CHUNK


def _moe_kernel(x_ref, gw_ref, wg_ref, wu_ref, wd_ref, out_ref, logits_ref,
                combine_ref):
    e = pl.program_id(0)
    c = pl.program_id(1)
    first = jnp.logical_and(e == 0, c == 0)

    @pl.when(first)
    def _router():
        x = x_ref[...]
        logits = jax.lax.dot_general(
            x, gw_ref[...], dimension_numbers=(((1,), (1,)), ((), ())),
            preferred_element_type=jnp.float32)
        logits_ref[...] = logits
        probs = jax.nn.softmax(logits, axis=-1)
        eidx = jax.lax.broadcasted_iota(jnp.int32, probs.shape, 1)
        p1 = jnp.max(probs, axis=-1, keepdims=True)
        i1 = jnp.min(jnp.where(probs >= p1, eidx, NUM_EXPERTS), axis=-1,
                     keepdims=True)
        sel1 = eidx == i1
        probs2 = jnp.where(sel1, -jnp.inf, probs)
        p2 = jnp.max(probs2, axis=-1, keepdims=True)
        i2 = jnp.min(jnp.where(probs2 >= p2, eidx, NUM_EXPERTS), axis=-1,
                     keepdims=True)
        sel2 = eidx == i2
        denom = p1 + p2
        combine_ref[...] = (jnp.where(sel1, p1, 0.0)
                            + jnp.where(sel2, p2, 0.0)) / denom

    x = x_ref[...]
    gate = jnp.dot(x, wg_ref[0], preferred_element_type=jnp.float32)
    up = jnp.dot(x, wu_ref[0], preferred_element_type=jnp.float32)
    hidden = gate * jax.nn.sigmoid(gate) * up
    down = jnp.dot(hidden, wd_ref[0], preferred_element_type=jnp.float32)
    combine = combine_ref[...]
    lane = jax.lax.broadcasted_iota(jnp.int32, combine.shape, 1)
    col = jnp.sum(jnp.where(lane == e, combine, 0.0), axis=-1, keepdims=True)
    contrib = col * down

    @pl.when(first)
    def _init():
        out_ref[...] = contrib

    @pl.when(jnp.logical_not(first))
    def _acc():
        out_ref[...] = out_ref[...] + contrib


@functools.partial(jax.jit, static_argnames=())
def kernel(hidden_states, gate_w, w_gate_up, w_down):
    b, s, d = hidden_states.shape
    t = b * s
    x = hidden_states.reshape(t, d)

    out, logits = pl.pallas_call(
        _moe_kernel,
        grid=(NUM_EXPERTS, N_CHUNKS),
        in_specs=[
            pl.BlockSpec((t, d), lambda e, c: (0, 0)),
            pl.BlockSpec((NUM_EXPERTS, d), lambda e, c: (0, 0)),
            pl.BlockSpec((1, d, CHUNK), lambda e, c: (e, 0, c)),
            pl.BlockSpec((1, d, CHUNK), lambda e, c: (e, 0, N_CHUNKS + c)),
            pl.BlockSpec((1, CHUNK, d), lambda e, c: (e, c, 0)),
        ],
        out_specs=[
            pl.BlockSpec((t, d), lambda e, c: (0, 0)),
            pl.BlockSpec((t, NUM_EXPERTS), lambda e, c: (0, 0)),
        ],
        out_shape=[
            jax.ShapeDtypeStruct((t, d), jnp.float32),
            jax.ShapeDtypeStruct((t, NUM_EXPERTS), jnp.float32),
        ],
        scratch_shapes=[pltpu.VMEM((t, NUM_EXPERTS), jnp.float32)],
        compiler_params=pltpu.CompilerParams(
            dimension_semantics=("arbitrary", "arbitrary"),
        ),
    )(x, gate_w, w_gate_up, w_gate_up, w_down)

    return out.reshape(b, s, d), logits
